# baseline (speedup 1.0000x reference)
import functools

import jax
import jax.numpy as jnp
from jax import lax
from jax.experimental import pallas as pl
from jax.experimental.pallas import tpu as pltpu
from jax.experimental.pallas import tpu_sc as plsc

E = 640000
H = 8
D = 16
N_NODES = 10000
EPS = 1e-8

HD = H * D
EP = E * H
EPQ = EP // 128
XW = 16 * HD

NW = 32
EPW = E // NW
TBL = 81920
SLICE = TBL // 16

C_SC = 1000
NCH_SC = EPW // C_SC
C_GA = 1000
NCH_GA = EPW // C_GA

_BLKF = 4000
_GRIDF = EPQ // _BLKF
_BLKN = 400
_GRIDN = EPQ // _BLKN


def _norm_body(xi_ref, xj_ref, ni_ref, sj_ref):
    kk = lax.broadcasted_iota(jnp.int32, (XW, 128), 0)
    ll = lax.broadcasted_iota(jnp.int32, (XW, 128), 1)
    w = ((kk // HD == ll // H) & (kk % HD // D == ll % H)).astype(jnp.float32)
    xi = xi_ref[...]
    ssqi = jnp.dot(xi * xi, w, preferred_element_type=jnp.float32)
    ni_ref[...] = jnp.sqrt(ssqi) + EPS
    xj = xj_ref[...]
    sj_ref[...] = jnp.dot(xj * xj, w, preferred_element_type=jnp.float32)


_norms = pl.pallas_call(
    _norm_body,
    grid=(_GRIDN,),
    in_specs=[
        pl.BlockSpec((_BLKN, XW), lambda i: (i, 0)),
        pl.BlockSpec((_BLKN, XW), lambda i: (i, 0)),
    ],
    out_specs=[
        pl.BlockSpec((_BLKN, 128), lambda i: (i, 0)),
        pl.BlockSpec((_BLKN, 128), lambda i: (i, 0)),
    ],
    out_shape=[
        jax.ShapeDtypeStruct((EPQ, 128), jnp.float32),
        jax.ShapeDtypeStruct((EPQ, 128), jnp.float32),
    ],
)


def _final_body(e_ref, ni_ref, g_ref, o_ref):
    den = 2.0 * (ni_ref[...] + g_ref[...]) + EPS
    r = e_ref[...] / den
    o_ref[...] = jnp.minimum(jnp.maximum(r, -10.0), 10.0)


_final = pl.pallas_call(
    _final_body,
    grid=(_GRIDF,),
    in_specs=[
        pl.BlockSpec((_BLKF, 128), lambda i: (i, 0)),
        pl.BlockSpec((_BLKF, 128), lambda i: (i, 0)),
        pl.BlockSpec((_BLKF, 128), lambda i: (i, 0)),
    ],
    out_specs=pl.BlockSpec((_BLKF, 128), lambda i: (i, 0)),
    out_shape=jax.ShapeDtypeStruct((EPQ, 128), jnp.float32),
)


def _sqrt16(s):
    i = plsc.bitcast(s, jnp.int32)
    i = 0x5F3759DF - (i >> 1)
    y = plsc.bitcast(i, jnp.float32)
    for _ in range(3):
        y = y * (1.5 - 0.5 * s * y * y)
    return s * y


_sc_mesh = plsc.VectorSubcoreMesh(core_axis_name="c", subcore_axis_name="s")
_sc_params = pltpu.CompilerParams(needs_layout_passes=False)


@functools.partial(
    pl.kernel,
    out_type=jax.ShapeDtypeStruct((NW, TBL), jnp.float32),
    mesh=_sc_mesh,
    scratch_types=[
        pltpu.VMEM((TBL,), jnp.float32),
        pltpu.VMEM((C_SC,), jnp.int32),
        pltpu.VMEM((C_SC * H,), jnp.float32),
    ],
    compiler_params=_sc_params,
)
def _scatter_max(idx_hbm, ssq_hbm, out_hbm, tbl, idx_v, val_v):
    cid = lax.axis_index("c")
    sid = lax.axis_index("s")
    wid = sid * 2 + cid

    zero = jnp.zeros((16,), jnp.float32)

    @pl.loop(0, TBL // 16)
    def _zero(i):
        tbl[pl.ds(i * 16, 16)] = zero

    iota = lax.iota(jnp.int32, 16)
    eoff = iota // H
    hh = iota - eoff * H

    base_e = wid * EPW

    @pl.loop(0, NCH_SC)
    def _chunk(ci):
        off = base_e + ci * C_SC
        pltpu.sync_copy(idx_hbm.at[pl.ds(off, C_SC)], idx_v)
        pltpu.sync_copy(ssq_hbm.at[pl.ds(off * H, C_SC * H)], val_v)

        @pl.loop(0, C_SC // 2)
        def _pair(j):
            e2 = plsc.load_gather(idx_v, [j * 2 + eoff])
            t = e2 * H + hh
            val = val_v[pl.ds(j * 16, 16)]
            cur = plsc.load_gather(tbl, [t])
            plsc.store_scatter(tbl, [t], jnp.maximum(cur, val))
            chk = plsc.load_gather(tbl, [t])
            lost = chk < val
            plsc.store_scatter(tbl, [t], jnp.maximum(chk, val), mask=lost)

    pltpu.sync_copy(tbl, out_hbm.at[wid])


@functools.partial(
    pl.kernel,
    out_type=jax.ShapeDtypeStruct((EP,), jnp.float32),
    mesh=_sc_mesh,
    scratch_types=[
        pltpu.VMEM((TBL,), jnp.float32),
        pltpu.VMEM((SLICE,), jnp.float32),
        pltpu.VMEM((SLICE,), jnp.float32),
        pltpu.VMEM_SHARED((TBL,), jnp.float32),
        pltpu.VMEM((C_GA,), jnp.int32),
        pltpu.VMEM((C_GA * H,), jnp.float32),
    ],
    compiler_params=_sc_params,
)
def _gather_tbl(idx_hbm, tbls_hbm, out_hbm, tblm, tmp_v, acc_v, stage,
                idx_v, o_v):
    cid = lax.axis_index("c")
    sid = lax.axis_index("s")
    wid = sid * 2 + cid

    mybase = sid * SLICE
    pltpu.sync_copy(tbls_hbm.at[0, pl.ds(mybase, SLICE)], acc_v)

    @pl.loop(1, NW)
    def _merge(t):
        pltpu.sync_copy(tbls_hbm.at[t, pl.ds(mybase, SLICE)], tmp_v)

        @pl.loop(0, SLICE // 16)
        def _mx(i):
            sl = pl.ds(i * 16, 16)
            acc_v[sl] = jnp.maximum(acc_v[sl], tmp_v[sl])

    @pl.loop(0, SLICE // 16)
    def _rt(i):
        sl = pl.ds(i * 16, 16)
        acc_v[sl] = _sqrt16(acc_v[sl]) + 2.0 * EPS

    pltpu.sync_copy(acc_v, stage.at[pl.ds(mybase, SLICE)])
    plsc.subcore_barrier()
    pltpu.sync_copy(stage, tblm)

    iota = lax.iota(jnp.int32, 16)
    eoff = iota // H
    hh = iota - eoff * H

    base_e = wid * EPW

    @pl.loop(0, NCH_GA)
    def _chunk(ci):
        off = base_e + ci * C_GA
        pltpu.sync_copy(idx_hbm.at[pl.ds(off, C_GA)], idx_v)

        @pl.loop(0, C_GA // 2)
        def _pair(j):
            e2 = plsc.load_gather(idx_v, [j * 2 + eoff])
            g = plsc.load_gather(tblm, [e2 * H + hh])
            o_v[pl.ds(j * 16, 16)] = g

        pltpu.sync_copy(o_v, out_hbm.at[pl.ds(off * H, C_GA * H)])


def kernel(e_ij, x_i, x_j, index):
    idx32 = index.astype(jnp.int32)
    ni_p, sj_p = _norms(x_i.reshape(EPQ, XW), x_j.reshape(EPQ, XW))
    tbls = _scatter_max(idx32, sj_p.reshape(EP))
    g = _gather_tbl(idx32, tbls)
    out = _final(e_ij.reshape(EPQ, 128), ni_p, g.reshape(EPQ, 128))
    return out.reshape(E, H)


# speedup vs baseline: 2.5445x; 2.5445x over previous
import functools

import jax
import jax.numpy as jnp
from jax import lax
from jax.experimental import pallas as pl
from jax.experimental.pallas import tpu as pltpu
from jax.experimental.pallas import tpu_sc as plsc

E = 640000
H = 8
D = 16
N_NODES = 10000
EPS = 1e-8

HD = H * D
EP = E * H
EPQ = EP // 128
XW = 16 * HD

NW = 32
EPW = E // NW
TBL = 81920
SLICE = TBL // 16

C_SC = 1000
NCH_SC = EPW // C_SC
C_GA = 1000
NCH_GA = EPW // C_GA

_BLKF = 4000
_GRIDF = EPQ // _BLKF
_BLKN = 400
_GRIDN = EPQ // _BLKN


def _norm_body(xi_ref, xj_ref, ni_ref, sj_ref):
    km = lax.broadcasted_iota(jnp.int32, (HD, H), 0) // D
    hm = lax.broadcasted_iota(jnp.int32, (HD, H), 1)
    m = (km == hm).astype(jnp.float32)
    xi = xi_ref[...]
    ssqi = jnp.dot(xi * xi, m, preferred_element_type=jnp.float32)
    ni_ref[...] = jnp.sqrt(ssqi) + EPS
    xj = xj_ref[...]
    sj_ref[...] = jnp.dot(xj * xj, m, preferred_element_type=jnp.float32)


_norms = pl.pallas_call(
    _norm_body,
    grid=(_GRIDN,),
    in_specs=[
        pl.BlockSpec((_BLKN * 16, HD), lambda i: (i, 0)),
        pl.BlockSpec((_BLKN * 16, HD), lambda i: (i, 0)),
    ],
    out_specs=[
        pl.BlockSpec((_BLKN * 16, H), lambda i: (i, 0)),
        pl.BlockSpec((_BLKN * 16, H), lambda i: (i, 0)),
    ],
    out_shape=[
        jax.ShapeDtypeStruct((E, H), jnp.float32),
        jax.ShapeDtypeStruct((E, H), jnp.float32),
    ],
)


def _final_body(e_ref, ni_ref, g_ref, o_ref):
    den = 2.0 * (ni_ref[...] + g_ref[...]) + EPS
    r = e_ref[...] / den
    o_ref[...] = jnp.minimum(jnp.maximum(r, -10.0), 10.0)


_final = pl.pallas_call(
    _final_body,
    grid=(_GRIDF,),
    in_specs=[
        pl.BlockSpec((_BLKF, 128), lambda i: (i, 0)),
        pl.BlockSpec((_BLKF, 128), lambda i: (i, 0)),
        pl.BlockSpec((_BLKF, 128), lambda i: (i, 0)),
    ],
    out_specs=pl.BlockSpec((_BLKF, 128), lambda i: (i, 0)),
    out_shape=jax.ShapeDtypeStruct((EPQ, 128), jnp.float32),
)


def _sqrt16(s):
    i = plsc.bitcast(s, jnp.int32)
    i = 0x5F3759DF - (i >> 1)
    y = plsc.bitcast(i, jnp.float32)
    for _ in range(3):
        y = y * (1.5 - 0.5 * s * y * y)
    return s * y


_sc_mesh = plsc.VectorSubcoreMesh(core_axis_name="c", subcore_axis_name="s")
_sc_params = pltpu.CompilerParams(needs_layout_passes=False)


@functools.partial(
    pl.kernel,
    out_type=jax.ShapeDtypeStruct((NW, TBL), jnp.float32),
    mesh=_sc_mesh,
    scratch_types=[
        pltpu.VMEM((TBL,), jnp.float32),
        pltpu.VMEM((C_SC,), jnp.int32),
        pltpu.VMEM((C_SC * H,), jnp.float32),
    ],
    compiler_params=_sc_params,
)
def _scatter_max(idx_hbm, ssq_hbm, out_hbm, tbl, idx_v, val_v):
    cid = lax.axis_index("c")
    sid = lax.axis_index("s")
    wid = sid * 2 + cid

    zero = jnp.zeros((16,), jnp.float32)

    @pl.loop(0, TBL // 16)
    def _zero(i):
        tbl[pl.ds(i * 16, 16)] = zero

    iota = lax.iota(jnp.int32, 16)
    eoff = iota // H
    hh = iota - eoff * H

    base_e = wid * EPW

    @pl.loop(0, NCH_SC)
    def _chunk(ci):
        off = base_e + ci * C_SC
        pltpu.sync_copy(idx_hbm.at[pl.ds(off, C_SC)], idx_v)
        pltpu.sync_copy(ssq_hbm.at[pl.ds(off * H, C_SC * H)], val_v)

        @pl.loop(0, C_SC // 2)
        def _pair(j):
            e2 = plsc.load_gather(idx_v, [j * 2 + eoff])
            t = e2 * H + hh
            val = val_v[pl.ds(j * 16, 16)]
            cur = plsc.load_gather(tbl, [t])
            plsc.store_scatter(tbl, [t], jnp.maximum(cur, val))
            chk = plsc.load_gather(tbl, [t])
            lost = chk < val
            plsc.store_scatter(tbl, [t], jnp.maximum(chk, val), mask=lost)

    pltpu.sync_copy(tbl, out_hbm.at[wid])


@functools.partial(
    pl.kernel,
    out_type=jax.ShapeDtypeStruct((EP,), jnp.float32),
    mesh=_sc_mesh,
    scratch_types=[
        pltpu.VMEM((TBL,), jnp.float32),
        pltpu.VMEM((SLICE,), jnp.float32),
        pltpu.VMEM((SLICE,), jnp.float32),
        pltpu.VMEM_SHARED((TBL,), jnp.float32),
        pltpu.VMEM((C_GA,), jnp.int32),
        pltpu.VMEM((C_GA * H,), jnp.float32),
    ],
    compiler_params=_sc_params,
)
def _gather_tbl(idx_hbm, tbls_hbm, out_hbm, tblm, tmp_v, acc_v, stage,
                idx_v, o_v):
    cid = lax.axis_index("c")
    sid = lax.axis_index("s")
    wid = sid * 2 + cid

    mybase = sid * SLICE
    pltpu.sync_copy(tbls_hbm.at[0, pl.ds(mybase, SLICE)], acc_v)

    @pl.loop(1, NW)
    def _merge(t):
        pltpu.sync_copy(tbls_hbm.at[t, pl.ds(mybase, SLICE)], tmp_v)

        @pl.loop(0, SLICE // 16)
        def _mx(i):
            sl = pl.ds(i * 16, 16)
            acc_v[sl] = jnp.maximum(acc_v[sl], tmp_v[sl])

    @pl.loop(0, SLICE // 16)
    def _rt(i):
        sl = pl.ds(i * 16, 16)
        acc_v[sl] = _sqrt16(acc_v[sl]) + 2.0 * EPS

    pltpu.sync_copy(acc_v, stage.at[pl.ds(mybase, SLICE)])
    plsc.subcore_barrier()
    pltpu.sync_copy(stage, tblm)

    iota = lax.iota(jnp.int32, 16)
    eoff = iota // H
    hh = iota - eoff * H

    base_e = wid * EPW

    @pl.loop(0, NCH_GA)
    def _chunk(ci):
        off = base_e + ci * C_GA
        pltpu.sync_copy(idx_hbm.at[pl.ds(off, C_GA)], idx_v)

        @pl.loop(0, C_GA // 2)
        def _pair(j):
            e2 = plsc.load_gather(idx_v, [j * 2 + eoff])
            g = plsc.load_gather(tblm, [e2 * H + hh])
            o_v[pl.ds(j * 16, 16)] = g

        pltpu.sync_copy(o_v, out_hbm.at[pl.ds(off * H, C_GA * H)])


def kernel(e_ij, x_i, x_j, index):
    idx32 = index.astype(jnp.int32)
    ni, sj = _norms(x_i.reshape(E, HD), x_j.reshape(E, HD))
    tbls = _scatter_max(idx32, sj.reshape(EP))
    g = _gather_tbl(idx32, tbls)
    out = _final(e_ij.reshape(EPQ, 128), ni.reshape(EPQ, 128),
                 g.reshape(EPQ, 128))
    return out.reshape(E, H)
